# Initial kernel scaffold; baseline (speedup 1.0000x reference)
import jax
import jax.numpy as jnp
from jax.experimental import pallas as pl


def kernel(x, edge_index, training, W1, b1, W2, b2, W3, b3):
    raise NotImplementedError("write your pallas kernel here")


# speedup vs baseline: 10.7555x; 10.7555x over previous
import functools

import jax
import jax.numpy as jnp
from jax import lax
from jax.experimental import pallas as pl
from jax.experimental.pallas import tpu as pltpu
from jax.experimental.pallas import tpu_sc as plsc

N = 10000
E = 320000
D = 128

_INFO = plsc.get_sparse_core_info()
NC = _INFO.num_cores
NS = _INFO.num_subcores
CHUNK = 80
EPT = E // (NC * NS)
NCHUNK = EPT // CHUNK
NPAD = 10240
RPT = NPAD // NS

_MESH = plsc.VectorSubcoreMesh(core_axis_name="c", subcore_axis_name="s")


def _edge_pass_body(hp_hbm, src_hbm, dst_hbm, zeros_hbm,
                    p0_hbm, p1_hbm,
                    src_idx, dst_idx, rows, sem, acc):
    c = lax.axis_index("c")
    s = lax.axis_index("s")
    wid = s * NC + c
    base = wid * EPT
    out_rows = pl.ds(s * RPT, RPT)

    pltpu.sync_copy(zeros_hbm.at[out_rows], acc.at[out_rows])
    plsc.subcore_barrier()

    def body(i, carry):
        eb = base + i * CHUNK
        pltpu.sync_copy(src_hbm.at[pl.ds(eb, CHUNK)], src_idx)
        pltpu.sync_copy(dst_hbm.at[pl.ds(eb, CHUNK)], dst_idx)
        pltpu.async_copy(hp_hbm.at[src_idx], rows, sem).wait()
        pltpu.sync_copy(rows, acc.at[dst_idx], add=True)
        return carry

    lax.fori_loop(0, NCHUNK, body, 0)
    plsc.subcore_barrier()

    @pl.when(c == 0)
    def _():
        pltpu.sync_copy(acc.at[out_rows], p0_hbm.at[out_rows])

    @pl.when(c == 1)
    def _():
        pltpu.sync_copy(acc.at[out_rows], p1_hbm.at[out_rows])


def _deg_pass_body(dst_hbm, zeros_hbm, ones_hbm,
                   p0_hbm, p1_hbm,
                   dst_idx, ones_v, sem, acc):
    c = lax.axis_index("c")
    s = lax.axis_index("s")
    wid = s * NC + c
    base = wid * EPT
    out_rows = pl.ds(s * RPT, RPT)

    pltpu.sync_copy(ones_hbm, ones_v)
    pltpu.sync_copy(zeros_hbm.at[out_rows], acc.at[out_rows])
    plsc.subcore_barrier()

    def body(i, carry):
        eb = base + i * CHUNK
        pltpu.sync_copy(dst_hbm.at[pl.ds(eb, CHUNK)], dst_idx)
        pltpu.sync_copy(ones_v, acc.at[dst_idx], add=True)
        return carry

    lax.fori_loop(0, NCHUNK, body, 0)
    plsc.subcore_barrier()

    @pl.when(c == 0)
    def _():
        pltpu.sync_copy(acc.at[out_rows], p0_hbm.at[out_rows])

    @pl.when(c == 1)
    def _():
        pltpu.sync_copy(acc.at[out_rows], p1_hbm.at[out_rows])


_edge_pass = functools.partial(
    pl.kernel, _edge_pass_body, mesh=_MESH,
    out_type=[jax.ShapeDtypeStruct((NPAD, D), jnp.float32),
              jax.ShapeDtypeStruct((NPAD, D), jnp.float32)],
    scratch_types=[pltpu.VMEM((CHUNK,), jnp.int32),
                   pltpu.VMEM((CHUNK,), jnp.int32),
                   pltpu.VMEM((CHUNK, D), jnp.float32),
                   pltpu.SemaphoreType.DMA,
                   pltpu.VMEM_SHARED((NPAD, D), jnp.float32)],
)()

_deg_pass = functools.partial(
    pl.kernel, _deg_pass_body, mesh=_MESH,
    out_type=[jax.ShapeDtypeStruct((NPAD, D), jnp.float32),
              jax.ShapeDtypeStruct((NPAD, D), jnp.float32)],
    scratch_types=[pltpu.VMEM((CHUNK,), jnp.int32),
                   pltpu.VMEM((CHUNK, D), jnp.float32),
                   pltpu.SemaphoreType.DMA,
                   pltpu.VMEM_SHARED((NPAD, D), jnp.float32)],
)()


_RB = 1000
_GRID = N // _RB


def _t_first(d0, d1, x, w, dis_out, hp_out):
    deg = d0[...] + d1[...] + 1.0
    dis = lax.rsqrt(deg)
    h = jnp.dot(x[...], w[...], preferred_element_type=jnp.float32)
    dis_out[...] = dis
    hp_out[...] = dis * h


def _t_mid(p0, p1, hp, dis, b, w, out):
    t = dis[...] * (p0[...] + p1[...] + hp[...]) + b[...]
    t = jnp.maximum(t, 0.0)
    h = jnp.dot(t, w[...], preferred_element_type=jnp.float32)
    out[...] = dis[...] * h


def _t_last(p0, p1, hp, dis, b, out):
    out[...] = dis[...] * (p0[...] + p1[...] + hp[...]) + b[...]


_row_spec = pl.BlockSpec((_RB, D), lambda i: (i, 0))
_mat_spec = pl.BlockSpec((D, D), lambda i: (0, 0))
_vec_spec = pl.BlockSpec((1, D), lambda i: (0, 0))
_row_out = jax.ShapeDtypeStruct((N, D), jnp.float32)

_t_first_call = pl.pallas_call(
    _t_first, grid=(_GRID,),
    in_specs=[_row_spec, _row_spec, _row_spec, _mat_spec],
    out_specs=[_row_spec, _row_spec],
    out_shape=[_row_out, _row_out],
)

_t_mid_call = pl.pallas_call(
    _t_mid, grid=(_GRID,),
    in_specs=[_row_spec, _row_spec, _row_spec, _row_spec, _vec_spec,
              _mat_spec],
    out_specs=_row_spec,
    out_shape=_row_out,
)

_t_last_call = pl.pallas_call(
    _t_last, grid=(_GRID,),
    in_specs=[_row_spec, _row_spec, _row_spec, _row_spec, _vec_spec],
    out_specs=_row_spec,
    out_shape=_row_out,
)


def kernel(x, edge_index, training, W1, b1, W2, b2, W3, b3):
    src = edge_index[0]
    dst = edge_index[1]
    zeros = jnp.zeros((NPAD, D), jnp.float32)
    ones = jnp.ones((CHUNK, D), jnp.float32)

    d0, d1 = _deg_pass(dst, zeros, ones)
    dis, hp = _t_first_call(d0, d1, x, W1)

    p0, p1 = _edge_pass(hp, src, dst, zeros)
    hp = _t_mid_call(p0, p1, hp, dis, b1.reshape(1, D), W2)

    p0, p1 = _edge_pass(hp, src, dst, zeros)
    hp = _t_mid_call(p0, p1, hp, dis, b2.reshape(1, D), W3)

    p0, p1 = _edge_pass(hp, src, dst, zeros)
    return _t_last_call(p0, p1, hp, dis, b3.reshape(1, D))


# speedup vs baseline: 23.1783x; 2.1550x over previous
import functools

import jax
import jax.numpy as jnp
from jax import lax
from jax.experimental import pallas as pl
from jax.experimental.pallas import tpu as pltpu
from jax.experimental.pallas import tpu_sc as plsc

N = 10000
E = 320000
D = 128

_INFO = plsc.get_sparse_core_info()
NC = _INFO.num_cores
NS = _INFO.num_subcores
CHUNK = 80
EPT = E // (NC * NS)
NCHUNK = EPT // CHUNK
NPAD = 10240
BLK = 25
NBLK = NCHUNK // BLK
RPT = NPAD // NS

_MESH = plsc.VectorSubcoreMesh(core_axis_name="c", subcore_axis_name="s")


def _edge_pass_body(hp_hbm, src_hbm, dst_hbm, zeros_hbm,
                    p0_hbm, p1_hbm,
                    src_idx, dst_idx, rows, sem, isem, acc):
    c = lax.axis_index("c")
    s = lax.axis_index("s")
    wid = s * NC + c
    out_rows = pl.ds(s * RPT, RPT)

    pltpu.sync_copy(src_hbm.at[wid, 0], src_idx.at[0])
    pltpu.sync_copy(dst_hbm.at[wid, 0], dst_idx.at[0])
    pltpu.sync_copy(zeros_hbm.at[out_rows], acc.at[out_rows])
    plsc.subcore_barrier()

    for k in range(NBLK):
        p = k % 2
        if k > 0:
            pltpu.make_async_copy(src_hbm.at[wid, k], src_idx.at[p],
                                  isem).wait()
            pltpu.make_async_copy(dst_hbm.at[wid, k], dst_idx.at[p],
                                  isem).wait()
        if k + 1 < NBLK:
            pltpu.async_copy(src_hbm.at[wid, k + 1],
                             src_idx.at[1 - p], isem)
            pltpu.async_copy(dst_hbm.at[wid, k + 1],
                             dst_idx.at[1 - p], isem)
        sb, db = src_idx.at[p], dst_idx.at[p]
        pltpu.async_copy(hp_hbm.at[sb.at[0]], rows.at[0], sem)

        def body(j, carry):
            for b in range(2):
                i = 2 * j + b
                pltpu.async_copy(hp_hbm.at[sb.at[i + 1]],
                                 rows.at[1 - b], sem)
                pltpu.make_async_copy(hp_hbm.at[sb.at[i]],
                                      rows.at[b], sem).wait()
                pltpu.sync_copy(rows.at[b], acc.at[db.at[i]], add=True)
            return carry

        lax.fori_loop(0, (BLK - 1) // 2, body, 0)
        pltpu.make_async_copy(hp_hbm.at[sb.at[BLK - 1]],
                              rows.at[0], sem).wait()
        pltpu.sync_copy(rows.at[0], acc.at[db.at[BLK - 1]], add=True)
    plsc.subcore_barrier()

    @pl.when(c == 0)
    def _():
        pltpu.sync_copy(acc.at[out_rows], p0_hbm.at[out_rows])

    @pl.when(c == 1)
    def _():
        pltpu.sync_copy(acc.at[out_rows], p1_hbm.at[out_rows])


def _deg_pass_body(dst_hbm, zeros_hbm, ones_hbm,
                   p0_hbm, p1_hbm,
                   dst_idx, ones_v, sem, acc):
    c = lax.axis_index("c")
    s = lax.axis_index("s")
    wid = s * NC + c
    out_rows = pl.ds(s * RPT, RPT)

    pltpu.sync_copy(ones_hbm, ones_v)
    pltpu.sync_copy(dst_hbm.at[wid], dst_idx)
    pltpu.sync_copy(zeros_hbm.at[out_rows], acc.at[out_rows])
    plsc.subcore_barrier()

    def body(k, i, carry):
        pltpu.sync_copy(ones_v, acc.at[dst_idx.at[k].at[i]], add=True)
        return carry

    for k in range(NBLK):
        lax.fori_loop(0, BLK, functools.partial(body, k), 0)
    plsc.subcore_barrier()

    @pl.when(c == 0)
    def _():
        pltpu.sync_copy(acc.at[out_rows], p0_hbm.at[out_rows])

    @pl.when(c == 1)
    def _():
        pltpu.sync_copy(acc.at[out_rows], p1_hbm.at[out_rows])


_edge_pass = functools.partial(
    pl.kernel, _edge_pass_body, mesh=_MESH,
    out_type=[jax.ShapeDtypeStruct((NPAD, D), jnp.float32),
              jax.ShapeDtypeStruct((NPAD, D), jnp.float32)],
    scratch_types=[pltpu.VMEM((2, BLK, CHUNK), jnp.int32),
                   pltpu.VMEM((2, BLK, CHUNK), jnp.int32),
                   pltpu.VMEM((2, CHUNK, D), jnp.float32),
                   pltpu.SemaphoreType.DMA,
                   pltpu.SemaphoreType.DMA,
                   pltpu.VMEM_SHARED((NPAD, D), jnp.float32)],
)()

_deg_pass = functools.partial(
    pl.kernel, _deg_pass_body, mesh=_MESH,
    out_type=[jax.ShapeDtypeStruct((NPAD, D), jnp.float32),
              jax.ShapeDtypeStruct((NPAD, D), jnp.float32)],
    scratch_types=[pltpu.VMEM((NBLK, BLK, CHUNK), jnp.int32),
                   pltpu.VMEM((CHUNK, D), jnp.float32),
                   pltpu.SemaphoreType.DMA,
                   pltpu.VMEM_SHARED((NPAD, D), jnp.float32)],
)()


_RB = 1000
_GRID = N // _RB


def _t_first(d0, d1, x, w, dis_out, hp_out):
    deg = d0[...] + d1[...] + 1.0
    dis = lax.rsqrt(deg)
    h = jnp.dot(x[...], w[...], preferred_element_type=jnp.float32)
    dis_out[...] = dis
    hp_out[...] = dis * h


def _t_mid(p0, p1, hp, dis, b, w, out):
    t = dis[...] * (p0[...] + p1[...] + hp[...]) + b[...]
    t = jnp.maximum(t, 0.0)
    h = jnp.dot(t, w[...], preferred_element_type=jnp.float32)
    out[...] = dis[...] * h


def _t_last(p0, p1, hp, dis, b, out):
    out[...] = dis[...] * (p0[...] + p1[...] + hp[...]) + b[...]


_row_spec = pl.BlockSpec((_RB, D), lambda i: (i, 0))
_mat_spec = pl.BlockSpec((D, D), lambda i: (0, 0))
_vec_spec = pl.BlockSpec((1, D), lambda i: (0, 0))
_row_out = jax.ShapeDtypeStruct((N, D), jnp.float32)

_t_first_call = pl.pallas_call(
    _t_first, grid=(_GRID,),
    in_specs=[_row_spec, _row_spec, _row_spec, _mat_spec],
    out_specs=[_row_spec, _row_spec],
    out_shape=[_row_out, _row_out],
)

_t_mid_call = pl.pallas_call(
    _t_mid, grid=(_GRID,),
    in_specs=[_row_spec, _row_spec, _row_spec, _row_spec, _vec_spec,
              _mat_spec],
    out_specs=_row_spec,
    out_shape=_row_out,
)

_t_last_call = pl.pallas_call(
    _t_last, grid=(_GRID,),
    in_specs=[_row_spec, _row_spec, _row_spec, _row_spec, _vec_spec],
    out_specs=_row_spec,
    out_shape=_row_out,
)


def kernel(x, edge_index, training, W1, b1, W2, b2, W3, b3):
    src = edge_index[0].reshape(NC * NS, NBLK, BLK, CHUNK)
    dst = edge_index[1].reshape(NC * NS, NBLK, BLK, CHUNK)
    zeros = jnp.zeros((NPAD, D), jnp.float32)
    ones = jnp.ones((CHUNK, D), jnp.float32)

    d0, d1 = _deg_pass(dst, zeros, ones)
    dis, hp = _t_first_call(d0, d1, x, W1)

    p0, p1 = _edge_pass(hp, src, dst, zeros)
    hp = _t_mid_call(p0, p1, hp, dis, b1.reshape(1, D), W2)

    p0, p1 = _edge_pass(hp, src, dst, zeros)
    hp = _t_mid_call(p0, p1, hp, dis, b2.reshape(1, D), W3)

    p0, p1 = _edge_pass(hp, src, dst, zeros)
    return _t_last_call(p0, p1, hp, dis, b3.reshape(1, D))
